# Initial kernel scaffold; baseline (speedup 1.0000x reference)
import jax
import jax.numpy as jnp
from jax.experimental import pallas as pl


def kernel(nodes, labels, levels, node_emd, taxo_emd, taxo_bias, transforms):
    raise NotImplementedError("write your pallas kernel here")


# speedup vs baseline: 3.9627x; 3.9627x over previous
import functools

import jax
import jax.numpy as jnp
from jax import lax
from jax.experimental import pallas as pl
from jax.experimental.pallas import tpu as pltpu
from jax.experimental.pallas import tpu_sc as plsc

_B = 16384
_E = 64
_L = 8
_AUG = 80
_LAMBDA = 1.0

try:
    _info = plsc.get_sparse_core_info()
    _NC, _NS = _info.num_cores, _info.num_subcores
except Exception:
    _NC, _NS = 2, 16
_NW = _NC * _NS
_BPW = _B // _NW
_CHUNK = 128
_NCHUNK = _BPW // _CHUNK


def _sc_gather(nid_hbm, cid_hbm, node_emd_hbm, aug_hbm,
               ne_out, aug_out, nidx, cidx, nrows, crows, sem_n, sem_c):
    wid = lax.axis_index("s") * _NC + lax.axis_index("c")
    base = wid * _BPW
    pltpu.sync_copy(nid_hbm.at[pl.ds(base, _BPW)], nidx)
    pltpu.sync_copy(cid_hbm.at[pl.ds(base, _BPW)], cidx)
    copies = []
    for j in range(_NCHUNK):
        sl = pl.ds(j * _CHUNK, _CHUNK)
        copies.append(pltpu.async_copy(
            node_emd_hbm.at[nidx.at[sl]], nrows.at[sl], sem_n))
        copies.append(pltpu.async_copy(
            aug_hbm.at[cidx.at[sl]], crows.at[sl], sem_c))
    for c in copies:
        c.wait()
    pltpu.sync_copy(nrows, ne_out.at[pl.ds(base, _BPW)])
    pltpu.sync_copy(crows, aug_out.at[pl.ds(base, _BPW)])


_sc_call_cache = []


def _sc_gather_call(*args):
    if not _sc_call_cache:
        _sc_call_cache.append(functools.partial(
            pl.kernel,
            mesh=plsc.VectorSubcoreMesh(core_axis_name="c",
                                        subcore_axis_name="s"),
            compiler_params=pltpu.CompilerParams(use_tc_tiling_on_sc=False),
            out_type=(
                jax.ShapeDtypeStruct((_B, _E), jnp.float32),
                jax.ShapeDtypeStruct((_B, _AUG), jnp.float32),
            ),
            scratch_types=[
                pltpu.VMEM((_BPW,), jnp.int32),
                pltpu.VMEM((_BPW,), jnp.int32),
                pltpu.VMEM((_BPW, _E), jnp.float32),
                pltpu.VMEM((_BPW, _AUG), jnp.float32),
                pltpu.SemaphoreType.DMA,
                pltpu.SemaphoreType.DMA,
            ],
        )(_sc_gather))
    return _sc_call_cache[0](*args)


_BLK = 1024
_GRID = _B // _BLK


def _tc_body(ne_ref, aug_ref, lv_ref, lb_ref, t_ref, out_ref):
    i = pl.program_id(0)
    ne = ne_ref[...]
    ce = aug_ref[:, :_E]
    bias = aug_ref[:, _E:_E + 1]
    lv = lv_ref[...]
    lb = lb_ref[...]
    acc = jnp.zeros((_BLK, 1), jnp.float32)
    for l in range(_L):
        tl = t_ref[l]
        norm = jnp.sqrt(jnp.sum(tl * tl, axis=0, keepdims=True))
        tln = tl / jnp.maximum(norm, 1e-12)
        p = jnp.dot(ne, tln, preferred_element_type=jnp.float32)
        s = jnp.sum(p * ce, axis=1, keepdims=True)
        acc = acc + jnp.where(lv == l, s, 0.0)
    score = acc + bias
    bce = (jnp.maximum(score, 0.0) - score * lb
           + jnp.log1p(jnp.exp(-jnp.abs(score))))
    bsum = jnp.sum(bce)

    @pl.when(i == 0)
    def _():
        out_ref[0, 0] = 0.0

    out_ref[0, 0] += bsum

    @pl.when(i == _GRID - 1)
    def _():
        out_ref[0, 0] = out_ref[0, 0] * (_LAMBDA / _B)


_tc_call = pl.pallas_call(
    _tc_body,
    grid=(_GRID,),
    in_specs=[
        pl.BlockSpec((_BLK, _E), lambda i: (i, 0)),
        pl.BlockSpec((_BLK, _AUG), lambda i: (i, 0)),
        pl.BlockSpec((_BLK, 1), lambda i: (i, 0)),
        pl.BlockSpec((_BLK, 1), lambda i: (i, 0)),
        pl.BlockSpec((_L, _E, _E), lambda i: (0, 0, 0)),
    ],
    out_specs=pl.BlockSpec(memory_space=pltpu.SMEM),
    out_shape=jax.ShapeDtypeStruct((1, 1), jnp.float32),
)


def kernel(nodes, labels, levels, node_emd, taxo_emd, taxo_bias, transforms):
    nid = nodes[:, 0].astype(jnp.int32)
    cid = nodes[:, 1].astype(jnp.int32)
    aug = jnp.concatenate(
        [taxo_emd, taxo_bias[:, None],
         jnp.zeros((taxo_emd.shape[0], _AUG - _E - 1), jnp.float32)], axis=1)
    ne, ce_aug = _sc_gather_call(nid, cid, node_emd, aug)
    lv2 = levels.astype(jnp.int32).reshape(_B, 1)
    lb2 = labels.reshape(_B, 1)
    out = _tc_call(ne, ce_aug, lv2, lb2, transforms)
    return out[0, 0]


# speedup vs baseline: 5.2247x; 1.3185x over previous
import functools

import jax
import jax.numpy as jnp
from jax import lax
from jax.experimental import pallas as pl
from jax.experimental.pallas import tpu as pltpu
from jax.experimental.pallas import tpu_sc as plsc

_B = 16384
_E = 64
_L = 8
_LAMBDA = 1.0

try:
    _info = plsc.get_sparse_core_info()
    _NC, _NS = _info.num_cores, _info.num_subcores
except Exception:
    _NC, _NS = 2, 16
_NW = _NC * _NS
_BPW = _B // _NW
_CHUNK = 128
_NCHUNK = _BPW // _CHUNK


def _sc_gather(nid_hbm, cid_hbm, node_emd_hbm, taxo_emd_hbm, bias_hbm,
               packed_out, bias_out,
               nidx, cidx, nrows, crows, brow, sem_n, sem_c, sem_b):
    wid = lax.axis_index("s") * _NC + lax.axis_index("c")
    base = wid * _BPW
    pltpu.sync_copy(nid_hbm.at[pl.ds(base, _BPW)], nidx)
    pltpu.sync_copy(cid_hbm.at[pl.ds(base, _BPW)], cidx)
    copies = []
    for j in range(_NCHUNK):
        sl = pl.ds(j * _CHUNK, _CHUNK)
        copies.append(pltpu.async_copy(
            node_emd_hbm.at[nidx.at[sl]], nrows.at[sl], sem_n))
        copies.append(pltpu.async_copy(
            taxo_emd_hbm.at[cidx.at[sl]], crows.at[sl], sem_c))
        copies.append(pltpu.async_copy(
            bias_hbm.at[cidx.at[sl]], brow.at[sl], sem_b))
    for c in copies:
        c.wait()
    pltpu.sync_copy(nrows, packed_out.at[pl.ds(base, _BPW), pl.ds(0, _E)])
    pltpu.sync_copy(crows, packed_out.at[pl.ds(base, _BPW), pl.ds(_E, _E)])
    pltpu.sync_copy(brow, bias_out.at[pl.ds(base, _BPW)])


_sc_call_cache = []


def _sc_gather_call(*args):
    if not _sc_call_cache:
        _sc_call_cache.append(functools.partial(
            pl.kernel,
            mesh=plsc.VectorSubcoreMesh(core_axis_name="c",
                                        subcore_axis_name="s"),
            compiler_params=pltpu.CompilerParams(use_tc_tiling_on_sc=False),
            out_type=(
                jax.ShapeDtypeStruct((_B, 2 * _E), jnp.float32),
                jax.ShapeDtypeStruct((_B,), jnp.float32),
            ),
            scratch_types=[
                pltpu.VMEM((_BPW,), jnp.int32),
                pltpu.VMEM((_BPW,), jnp.int32),
                pltpu.VMEM((_BPW, _E), jnp.float32),
                pltpu.VMEM((_BPW, _E), jnp.float32),
                pltpu.VMEM((_BPW,), jnp.float32),
                pltpu.SemaphoreType.DMA,
                pltpu.SemaphoreType.DMA,
                pltpu.SemaphoreType.DMA,
            ],
        )(_sc_gather))
    return _sc_call_cache[0](*args)


_BLK = 1024
_GRID = _B // _BLK


def _tc_body(pk_ref, bias_ref, lv_ref, lb_ref, t_ref, out_ref):
    i = pl.program_id(0)
    f32 = jnp.float32
    rows = lax.broadcasted_iota(jnp.int32, (2 * _E, 2 * _E), 0)
    cols = lax.broadcasted_iota(jnp.int32, (2 * _E, 2 * _E), 1)
    ident = (rows == cols).astype(f32)
    i64 = ident[:_E, :_E]
    dn = (((1,), (1,)), ((), ()))
    pk_t = lax.dot_general(ident, pk_ref[...], dn,
                           preferred_element_type=f32)
    ne_t = pk_t[:_E]
    ce_t = pk_t[_E:]
    lv = lv_ref[...]
    lb = lb_ref[...]
    bias = bias_ref[...]

    wcat = []
    necat = []
    for l in range(_L):
        tl = t_ref[l]
        norm = jnp.sqrt(jnp.sum(tl * tl, axis=0, keepdims=True))
        tln = tl / jnp.maximum(norm, 1e-12)
        wcat.append(lax.dot_general(i64, tln, dn,
                                    preferred_element_type=f32))
        necat.append(jnp.where(lv == l, ne_t, 0.0))
    wcat_t = jnp.concatenate(wcat, axis=1)
    necat_t = jnp.concatenate(necat, axis=0)
    z_t = jnp.dot(wcat_t, necat_t, preferred_element_type=f32)
    s = jnp.sum(z_t * ce_t, axis=0, keepdims=True)
    score = s + bias
    bce = (jnp.maximum(score, 0.0) - score * lb
           + jnp.log1p(jnp.exp(-jnp.abs(score))))
    bsum = jnp.sum(bce)

    @pl.when(i == 0)
    def _():
        out_ref[0, 0] = 0.0

    out_ref[0, 0] += bsum

    @pl.when(i == _GRID - 1)
    def _():
        out_ref[0, 0] = out_ref[0, 0] * (_LAMBDA / _B)


_tc_call = pl.pallas_call(
    _tc_body,
    grid=(_GRID,),
    in_specs=[
        pl.BlockSpec((_BLK, 2 * _E), lambda i: (i, 0)),
        pl.BlockSpec((1, _BLK), lambda i: (0, i)),
        pl.BlockSpec((1, _BLK), lambda i: (0, i)),
        pl.BlockSpec((1, _BLK), lambda i: (0, i)),
        pl.BlockSpec((_L, _E, _E), lambda i: (0, 0, 0)),
    ],
    out_specs=pl.BlockSpec(memory_space=pltpu.SMEM),
    out_shape=jax.ShapeDtypeStruct((1, 1), jnp.float32),
)


def kernel(nodes, labels, levels, node_emd, taxo_emd, taxo_bias, transforms):
    nid = nodes[:, 0].astype(jnp.int32)
    cid = nodes[:, 1].astype(jnp.int32)
    packed, biasg = _sc_gather_call(nid, cid, node_emd, taxo_emd, taxo_bias)
    lv2 = levels.astype(jnp.int32).reshape(1, _B)
    lb2 = labels.reshape(1, _B)
    bias2 = biasg.reshape(1, _B)
    out = _tc_call(packed, bias2, lv2, lb2, transforms)
    return out[0, 0]
